# baseline (speedup 1.0000x reference)
import jax
import jax.numpy as jnp
from jax import lax
from jax.experimental import pallas as pl
from jax.experimental.pallas import tpu as pltpu
from jax.experimental.pallas import tpu_sc as plsc

_NC, _NS, _LANES = 1, 16, 16
_NW = _NC * _NS


def kernel(input, embed_weight):
    L = input.shape[1]
    D = embed_weight.shape[1]
    per_w = L // _NW
    groups = per_w // _LANES
    idx = input.reshape(L).astype(jnp.int32)

    mesh = plsc.VectorSubcoreMesh(
        core_axis_name="c", subcore_axis_name="s", num_cores=1
    )

    @pl.kernel(
        out_type=jax.ShapeDtypeStruct((L * D,), embed_weight.dtype),
        mesh=mesh,
        compiler_params=pltpu.CompilerParams(
            needs_layout_passes=False, skip_device_barrier=True
        ),
        scratch_types=[
            pltpu.VMEM(embed_weight.shape, embed_weight.dtype),
            pltpu.VMEM((per_w,), jnp.int32),
            pltpu.VMEM((per_w * D,), embed_weight.dtype),
            pltpu.SemaphoreType.DMA,
            pltpu.SemaphoreType.DMA,
        ],
    )
    def _embed_kernel(table_hbm, idx_hbm, out_hbm, table_v, idx_v, out_v,
                      sem_t, sem_i):
        wid = lax.axis_index("s") * _NC + lax.axis_index("c")
        c_t = pltpu.async_copy(table_hbm, table_v, sem_t)
        c_i = pltpu.async_copy(idx_hbm.at[pl.ds(wid * per_w, per_w)], idx_v,
                               sem_i)
        lanes = lax.iota(jnp.int32, _LANES)
        s_pat = [lanes * D + d for d in range(D)]
        d_vec = [jnp.full((_LANES,), d, jnp.int32) for d in range(D)]
        c_i.wait()
        c_t.wait()

        @pl.loop(0, groups)
        def _(g):
            rows = idx_v[pl.ds(g * _LANES, _LANES)]
            base = g * (_LANES * D)
            for d in range(D):
                vals = plsc.load_gather(table_v, [rows, d_vec[d]])
                plsc.store_scatter(out_v, [s_pat[d] + base], vals)

        pltpu.sync_copy(out_v, out_hbm.at[pl.ds(wid * per_w * D, per_w * D)])

    return _embed_kernel(embed_weight, idx).reshape(1, L, D)


# speedup vs baseline: 1.0362x; 1.0362x over previous
import jax
import jax.numpy as jnp
from jax import lax
from jax.experimental import pallas as pl
from jax.experimental.pallas import tpu as pltpu
from jax.experimental.pallas import tpu_sc as plsc

_NC, _NS, _LANES = 2, 16, 16
_NW = _NC * _NS


def kernel(input, embed_weight):
    L = input.shape[1]
    D = embed_weight.shape[1]
    per_w = L // _NW
    groups = per_w // _LANES
    idx = input.reshape(L).astype(jnp.int32)

    mesh = plsc.VectorSubcoreMesh(core_axis_name="c", subcore_axis_name="s")

    @pl.kernel(
        out_type=jax.ShapeDtypeStruct((L * D,), embed_weight.dtype),
        mesh=mesh,
        compiler_params=pltpu.CompilerParams(
            needs_layout_passes=False, skip_device_barrier=True
        ),
        scratch_types=[
            pltpu.VMEM(embed_weight.shape, embed_weight.dtype),
            pltpu.VMEM((per_w,), jnp.int32),
            pltpu.VMEM((per_w * D,), embed_weight.dtype),
            pltpu.SemaphoreType.DMA,
            pltpu.SemaphoreType.DMA,
        ],
    )
    def _embed_kernel(table_hbm, idx_hbm, out_hbm, table_v, idx_v, out_v,
                      sem_t, sem_i):
        wid = lax.axis_index("s") * _NC + lax.axis_index("c")
        c_t = pltpu.async_copy(table_hbm, table_v, sem_t)
        c_i = pltpu.async_copy(idx_hbm.at[pl.ds(wid * per_w, per_w)], idx_v,
                               sem_i)
        lanes = lax.iota(jnp.int32, _LANES)
        s_pat = [lanes * D + d for d in range(D)]
        d_vec = [jnp.full((_LANES,), d, jnp.int32) for d in range(D)]
        c_i.wait()
        c_t.wait()

        @pl.loop(0, groups)
        def _(g):
            rows = idx_v[pl.ds(g * _LANES, _LANES)]
            base = g * (_LANES * D)
            for d in range(D):
                vals = plsc.load_gather(table_v, [rows, d_vec[d]])
                plsc.store_scatter(out_v, [s_pat[d] + base], vals)

        pltpu.sync_copy(out_v, out_hbm.at[pl.ds(wid * per_w * D, per_w * D)])

    return _embed_kernel(embed_weight, idx).reshape(1, L, D)


# speedup vs baseline: 1.2496x; 1.2060x over previous
import jax
import jax.numpy as jnp
from jax import lax
from jax.experimental import pallas as pl
from jax.experimental.pallas import tpu as pltpu
from jax.experimental.pallas import tpu_sc as plsc

_NC, _NS, _LANES = 2, 16, 16
_NW = _NC * _NS


def kernel(input, embed_weight):
    L = input.shape[1]
    D = embed_weight.shape[1]
    per_w = L // _NW
    groups = per_w // _LANES
    idx = input.astype(jnp.int32)

    mesh = plsc.VectorSubcoreMesh(core_axis_name="c", subcore_axis_name="s")

    @pl.kernel(
        out_type=jax.ShapeDtypeStruct((1, L, D), embed_weight.dtype),
        mesh=mesh,
        compiler_params=pltpu.CompilerParams(
            needs_layout_passes=False, skip_device_barrier=True
        ),
        scratch_types=[
            pltpu.VMEM(embed_weight.shape, embed_weight.dtype),
            pltpu.VMEM((per_w,), jnp.int32),
            pltpu.VMEM((per_w, D), embed_weight.dtype),
            pltpu.SemaphoreType.DMA,
            pltpu.SemaphoreType.DMA,
        ],
    )
    def _embed_kernel(table_hbm, idx_hbm, out_hbm, table_v, idx_v, out_v,
                      sem_t, sem_i):
        wid = lax.axis_index("s") * _NC + lax.axis_index("c")
        base = wid * per_w
        c_t = pltpu.async_copy(table_hbm, table_v, sem_t)
        c_i = pltpu.async_copy(idx_hbm.at[0, pl.ds(base, per_w)], idx_v, sem_i)
        lanes = lax.iota(jnp.int32, _LANES)
        d_vec = [jnp.full((_LANES,), d, jnp.int32) for d in range(D)]
        c_i.wait()
        c_t.wait()

        @pl.loop(0, groups)
        def _(g):
            rows = idx_v[pl.ds(g * _LANES, _LANES)]
            ivec = lanes + g * _LANES
            for d in range(D):
                vals = plsc.load_gather(table_v, [rows, d_vec[d]])
                plsc.store_scatter(out_v, [ivec, d_vec[d]], vals)

        pltpu.sync_copy(out_v, out_hbm.at[0, pl.ds(base, per_w)])

    return _embed_kernel(embed_weight, idx)


# speedup vs baseline: 1.7448x; 1.3963x over previous
import jax
import jax.numpy as jnp
from jax import lax
from jax.experimental import pallas as pl
from jax.experimental.pallas import tpu as pltpu
from jax.experimental.pallas import tpu_sc as plsc

_NC, _NS, _LANES = 2, 16, 16
_NW = _NC * _NS


def kernel(input, embed_weight):
    L = input.shape[1]
    D = embed_weight.shape[1]
    per_w = L // _NW
    groups = per_w // _LANES
    idx = input.astype(jnp.int32)

    mesh = plsc.VectorSubcoreMesh(core_axis_name="c", subcore_axis_name="s")

    @pl.kernel(
        out_type=jax.ShapeDtypeStruct((D, 1, L), embed_weight.dtype),
        mesh=mesh,
        compiler_params=pltpu.CompilerParams(
            needs_layout_passes=False, skip_device_barrier=True
        ),
        scratch_types=[
            pltpu.VMEM(embed_weight.shape, embed_weight.dtype),
            pltpu.VMEM((per_w,), jnp.int32),
            pltpu.VMEM((D * per_w,), embed_weight.dtype),
            pltpu.SemaphoreType.DMA,
            pltpu.SemaphoreType.DMA,
        ],
    )
    def _embed_kernel(table_hbm, idx_hbm, out_hbm, table_v, idx_v, out_v,
                      sem_t, sem_i):
        wid = lax.axis_index("s") * _NC + lax.axis_index("c")
        base = wid * per_w
        c_t = pltpu.async_copy(table_hbm, table_v, sem_t)
        c_i = pltpu.async_copy(idx_hbm.at[0, pl.ds(base, per_w)], idx_v, sem_i)
        d_vec = [jnp.full((_LANES,), d, jnp.int32) for d in range(D)]
        c_i.wait()
        c_t.wait()

        @pl.loop(0, groups)
        def _(g):
            rows = idx_v[pl.ds(g * _LANES, _LANES)]
            for d in range(D):
                vals = plsc.load_gather(table_v, [rows, d_vec[d]])
                out_v[pl.ds(d * per_w + g * _LANES, _LANES)] = vals

        out_copies = [
            pltpu.async_copy(out_v.at[pl.ds(d * per_w, per_w)],
                             out_hbm.at[d, 0, pl.ds(base, per_w)],
                             sem_t)
            for d in range(D)
        ]
        for c in out_copies:
            c.wait()

    return jnp.transpose(_embed_kernel(embed_weight, idx), (1, 2, 0))


# speedup vs baseline: 1.7476x; 1.0016x over previous
import jax
import jax.numpy as jnp
from jax import lax
from jax.experimental import pallas as pl
from jax.experimental.pallas import tpu as pltpu
from jax.experimental.pallas import tpu_sc as plsc

_NC, _NS, _LANES = 2, 16, 16
_NW = _NC * _NS


def kernel(input, embed_weight):
    L = input.shape[1]
    D = embed_weight.shape[1]
    per_w = L // _NW
    groups = per_w // _LANES
    idx = input.astype(jnp.int32)

    mesh = plsc.VectorSubcoreMesh(core_axis_name="c", subcore_axis_name="s")

    @pl.kernel(
        out_type=jax.ShapeDtypeStruct((D, 1, L), embed_weight.dtype),
        mesh=mesh,
        compiler_params=pltpu.CompilerParams(
            needs_layout_passes=False, skip_device_barrier=True
        ),
        scratch_types=[
            pltpu.VMEM(embed_weight.shape, embed_weight.dtype),
            pltpu.VMEM((per_w,), jnp.int32),
            pltpu.VMEM((D * per_w,), embed_weight.dtype),
            pltpu.SemaphoreType.DMA,
            pltpu.SemaphoreType.DMA,
        ],
    )
    def _embed_kernel(table_hbm, idx_hbm, out_hbm, table_v, idx_v, out_v,
                      sem_t, sem_i):
        wid = lax.axis_index("s") * _NC + lax.axis_index("c")
        base = wid * per_w
        c_t = pltpu.async_copy(table_hbm, table_v, sem_t)
        c_i = pltpu.async_copy(idx_hbm.at[0, pl.ds(base, per_w)], idx_v, sem_i)
        d_vec = [jnp.full((_LANES,), d, jnp.int32) for d in range(D)]
        c_i.wait()
        c_t.wait()

        @pl.loop(0, groups, step=2)
        def _(g):
            for k in range(2):
                rows = idx_v[pl.ds((g + k) * _LANES, _LANES)]
                for d in range(D):
                    vals = plsc.load_gather(table_v, [rows, d_vec[d]])
                    out_v[pl.ds(d * per_w + (g + k) * _LANES, _LANES)] = vals

        out_copies = [
            pltpu.async_copy(out_v.at[pl.ds(d * per_w, per_w)],
                             out_hbm.at[d, 0, pl.ds(base, per_w)],
                             sem_t)
            for d in range(D)
        ]
        for c in out_copies:
            c.wait()

    return jnp.transpose(_embed_kernel(embed_weight, idx), (1, 2, 0))
